# baseline (speedup 1.0000x reference)
import functools

import jax
import jax.numpy as jnp
from jax import lax
from jax.experimental import pallas as pl
from jax.experimental.pallas import tpu as pltpu
from jax.experimental.pallas import tpu_sc as plsc

N_ATOMS = 10000
N_BONDS = 160000
MAX_NB = 16
ATOM_FDIM = 128
BOND_FDIM = 144
H = 256
DEPTH = 4
N_MOLS = 100
APM = 100

NC, NS = 2, 16
NW = NC * NS
E_PAD = 163840
N_PAD = 10240
BPW = E_PAD // NW
APW = N_PAD // NW
CB = 128
CA = 8
LC = H // 16
LCB = H // 32

_MESH = dict(core_axis_name="c", subcore_axis_name="s")


def _mm_in_body(x_ref, wi_ref, wh_ref, inp_ref, m2_ref):
    inp = jnp.dot(x_ref[...], wi_ref[...], preferred_element_type=jnp.float32)
    inp_ref[...] = inp
    m = jnp.maximum(inp, 0.0).astype(jnp.bfloat16)
    m2_ref[...] = jnp.dot(m, wh_ref[...],
                          preferred_element_type=jnp.float32).astype(jnp.bfloat16)


def _mm_in(fb, wi, wh_b):
    RB = 2048
    return pl.pallas_call(
        _mm_in_body,
        grid=(E_PAD // RB,),
        in_specs=[pl.BlockSpec((RB, BOND_FDIM), lambda i: (i, 0)),
                  pl.BlockSpec((BOND_FDIM, H), lambda i: (0, 0)),
                  pl.BlockSpec((H, H), lambda i: (0, 0))],
        out_specs=[pl.BlockSpec((RB, H), lambda i: (i, 0)),
                   pl.BlockSpec((RB, H), lambda i: (i, 0))],
        out_shape=[jax.ShapeDtypeStruct((E_PAD, H), jnp.float32),
                   jax.ShapeDtypeStruct((E_PAD, H), jnp.bfloat16)],
    )(fb, wi, wh_b)


def _mm_h_body(inp_ref, g_ref, wh_ref, m2_ref):
    m = jnp.maximum(inp_ref[...] + g_ref[...].astype(jnp.float32), 0.0)
    m2_ref[...] = jnp.dot(m.astype(jnp.bfloat16), wh_ref[...],
                          preferred_element_type=jnp.float32).astype(jnp.bfloat16)


def _mm_h(inp, g, wh_b):
    RB = 2048
    return pl.pallas_call(
        _mm_h_body,
        grid=(E_PAD // RB,),
        in_specs=[pl.BlockSpec((RB, H), lambda i: (i, 0)),
                  pl.BlockSpec((RB, H), lambda i: (i, 0)),
                  pl.BlockSpec((H, H), lambda i: (0, 0))],
        out_specs=pl.BlockSpec((RB, H), lambda i: (i, 0)),
        out_shape=jax.ShapeDtypeStruct((E_PAD, H), jnp.bfloat16),
    )(inp, g, wh_b)


def _relu_add_body(inp_ref, g_ref, out_ref):
    out_ref[...] = jnp.maximum(inp_ref[...] + g_ref[...].astype(jnp.float32), 0.0)


def _relu_add(inp, g):
    RB = 4096
    return pl.pallas_call(
        _relu_add_body,
        grid=(E_PAD // RB,),
        in_specs=[pl.BlockSpec((RB, H), lambda i: (i, 0)),
                  pl.BlockSpec((RB, H), lambda i: (i, 0))],
        out_specs=pl.BlockSpec((RB, H), lambda i: (i, 0)),
        out_shape=jax.ShapeDtypeStruct((E_PAD, H), jnp.float32),
    )(inp, g)


def _final_body(fa_ref, am_ref, wo_ref, bo_ref, out_ref):
    wo = wo_ref[...]
    h = jnp.dot(fa_ref[...], wo[:ATOM_FDIM], preferred_element_type=jnp.float32)
    h = h + jnp.dot(am_ref[...], wo[ATOM_FDIM:], preferred_element_type=jnp.float32)
    h = jnp.maximum(h + bo_ref[...], 0.0)
    r = lax.broadcasted_iota(jnp.int32, (N_MOLS, N_PAD), 1) // APM
    m = lax.broadcasted_iota(jnp.int32, (N_MOLS, N_PAD), 0)
    sel = jnp.where(r == m, 1.0 / APM, 0.0)
    out_ref[...] = jnp.dot(sel, h, preferred_element_type=jnp.float32)


def _final(fa, am, wo, bo2):
    return pl.pallas_call(
        _final_body,
        out_shape=jax.ShapeDtypeStruct((N_MOLS, H), jnp.float32),
    )(fa, am, wo, bo2)


_M_HI = -65536
_M_LO = 0xFFFF
_RND = 0x7FFF
_ONE = 1


def _unpack16(w):
    lo = lax.bitcast_convert_type(w << 16, jnp.float32)
    hi = lax.bitcast_convert_type(w & _M_HI, jnp.float32)
    return lo, hi


def _rne(b):
    return b + _RND + ((lax.shift_right_logical(b, 16)) & _ONE)


def _pack16(lo, hi):
    lb = _rne(lax.bitcast_convert_type(lo, jnp.int32))
    hb = _rne(lax.bitcast_convert_type(hi, jnp.int32))
    return (hb & _M_HI) | (lax.shift_right_logical(lb, 16) & _M_LO)


@functools.partial(
    pl.kernel,
    mesh=plsc.VectorSubcoreMesh(**_MESH),
    out_type=jax.ShapeDtypeStruct((N_PAD, 128), jnp.int32),
    scratch_types=[
        pltpu.VMEM((CA * MAX_NB,), jnp.int32),
        pltpu.VMEM((CA * MAX_NB, 128), jnp.int32),
        pltpu.VMEM((CA, 128), jnp.int32),
        pltpu.SemaphoreType.DMA,
    ],
)
def _segsum_bf(a2b_hbm, m2_hbm, out_hbm, idx_v, rows_v, acc_v, sem):
    wid = lax.axis_index("s") * NC + lax.axis_index("c")
    base = wid * APW

    def chunk(ci, _):
        a0 = base + ci * CA
        pltpu.sync_copy(a2b_hbm.at[pl.ds(a0 * MAX_NB, CA * MAX_NB)], idx_v)
        pltpu.async_copy(m2_hbm.at[idx_v], rows_v, sem).wait()

        def atom(a, _):
            r0 = a * MAX_NB
            for jj in range(8):
                j = jj * 16
                acc0, acc1 = _unpack16(rows_v[r0, pl.ds(j, 16)])
                for k in range(1, MAX_NB):
                    x0, x1 = _unpack16(rows_v[r0 + k, pl.ds(j, 16)])
                    acc0 = acc0 + x0
                    acc1 = acc1 + x1
                acc_v[a, pl.ds(j, 16)] = _pack16(acc0, acc1)
            return 0

        lax.fori_loop(0, CA, atom, 0)
        pltpu.sync_copy(acc_v, out_hbm.at[pl.ds(a0, CA)])
        return 0

    lax.fori_loop(0, APW // CA, chunk, 0)


@functools.partial(
    pl.kernel,
    mesh=plsc.VectorSubcoreMesh(**_MESH),
    out_type=jax.ShapeDtypeStruct((N_PAD, H), jnp.float32),
    scratch_types=[
        pltpu.VMEM((CA * MAX_NB,), jnp.int32),
        pltpu.VMEM((CA * MAX_NB, H), jnp.float32),
        pltpu.VMEM((CA, H), jnp.float32),
        pltpu.SemaphoreType.DMA,
    ],
)
def _segsum_f32(a2b_hbm, msg_hbm, out_hbm, idx_v, rows_v, acc_v, sem):
    wid = lax.axis_index("s") * NC + lax.axis_index("c")
    base = wid * APW

    def chunk(ci, _):
        a0 = base + ci * CA
        pltpu.sync_copy(a2b_hbm.at[pl.ds(a0 * MAX_NB, CA * MAX_NB)], idx_v)
        pltpu.async_copy(msg_hbm.at[idx_v], rows_v, sem).wait()

        def atom(a, _):
            r0 = a * MAX_NB
            for jj in range(LC):
                j = jj * 16
                acc = rows_v[r0, pl.ds(j, 16)]
                for k in range(1, MAX_NB):
                    acc = acc + rows_v[r0 + k, pl.ds(j, 16)]
                acc_v[a, pl.ds(j, 16)] = acc
            return 0

        lax.fori_loop(0, CA, atom, 0)
        pltpu.sync_copy(acc_v, out_hbm.at[pl.ds(a0, CA)])
        return 0

    lax.fori_loop(0, APW // CA, chunk, 0)


@functools.partial(
    pl.kernel,
    mesh=plsc.VectorSubcoreMesh(**_MESH),
    out_type=jax.ShapeDtypeStruct((E_PAD, 128), jnp.int32),
    scratch_types=[
        pltpu.VMEM((CB,), jnp.int32),
        pltpu.VMEM((CB,), jnp.int32),
        pltpu.VMEM((CB, 128), jnp.int32),
        pltpu.VMEM((CB, 128), jnp.int32),
        pltpu.SemaphoreType.DMA,
        pltpu.SemaphoreType.DMA,
    ],
)
def _gather_sub(b2a_hbm, b2revb_hbm, amsg_hbm, m2_hbm, out_hbm,
                idxa_v, idxr_v, ga_v, gr_v, sema, semr):
    wid = lax.axis_index("s") * NC + lax.axis_index("c")
    base = wid * BPW

    def chunk(ci, _):
        e0 = base + ci * CB
        pltpu.sync_copy(b2a_hbm.at[pl.ds(e0, CB)], idxa_v)
        pltpu.sync_copy(b2revb_hbm.at[pl.ds(e0, CB)], idxr_v)
        cpa = pltpu.async_copy(amsg_hbm.at[idxa_v], ga_v, sema)
        cpr = pltpu.async_copy(m2_hbm.at[idxr_v], gr_v, semr)
        cpa.wait()
        cpr.wait()

        def row(r, _):
            for jj in range(8):
                j = jj * 16
                a0, a1 = _unpack16(ga_v[r, pl.ds(j, 16)])
                b0, b1 = _unpack16(gr_v[r, pl.ds(j, 16)])
                ga_v[r, pl.ds(j, 16)] = _pack16(a0 - b0, a1 - b1)
            return 0

        lax.fori_loop(0, CB, row, 0)
        pltpu.sync_copy(ga_v, out_hbm.at[pl.ds(e0, CB)])
        return 0

    lax.fori_loop(0, BPW // CB, chunk, 0)


def kernel(f_atoms, f_bonds, a2b, b2a, b2revb, W_i, W_h, W_o, b_o):
    a2b_flat = jnp.pad(a2b.astype(jnp.int32).reshape(-1),
                       (0, (N_PAD - N_ATOMS) * MAX_NB))
    b2a_p = jnp.pad(b2a.astype(jnp.int32), (0, E_PAD - N_BONDS))
    b2revb_p = jnp.pad(b2revb.astype(jnp.int32), (0, E_PAD - N_BONDS))
    fb_p = jnp.pad(f_bonds, ((0, E_PAD - N_BONDS), (0, 0)))
    fa_p = jnp.pad(f_atoms, ((0, N_PAD - N_ATOMS), (0, 0)))
    wh_b = W_h.astype(jnp.bfloat16)

    def pack(x_bf):
        return jax.lax.bitcast_convert_type(
            x_bf.reshape(x_bf.shape[0], H // 2, 2), jnp.int32)

    def unpack(x_i32):
        return jax.lax.bitcast_convert_type(
            x_i32, jnp.bfloat16).reshape(x_i32.shape[0], H)

    inp, m2 = _mm_in(fb_p, W_i, wh_b)
    m2p = pack(m2)
    g = None
    for it in range(DEPTH - 1):
        amsg2 = _segsum_bf(a2b_flat, m2p)
        g = unpack(_gather_sub(b2a_p, b2revb_p, amsg2, m2p))
        if it < DEPTH - 2:
            m2p = pack(_mm_h(inp, g, wh_b))
    msg = _relu_add(inp, g)
    amsg = _segsum_f32(a2b_flat, msg)
    return _final(fa_p, amsg, W_o, b_o.reshape(1, H))


# speedup vs baseline: 2.3329x; 2.3329x over previous
import functools

import jax
import jax.numpy as jnp
from jax import lax
from jax.experimental import pallas as pl
from jax.experimental.pallas import tpu as pltpu
from jax.experimental.pallas import tpu_sc as plsc

N_ATOMS = 10000
N_BONDS = 160000
MAX_NB = 16
ATOM_FDIM = 128
BOND_FDIM = 144
H = 256
DEPTH = 4
N_MOLS = 100
APM = 100

NC, NS = 2, 16
NW = NC * NS
E_PAD = 163840
N_PAD = 10240
BPW = E_PAD // NW
APW = N_PAD // NW
CB = 128
CA = 8
LC = H // 16
LCB = H // 32

_MESH = dict(core_axis_name="c", subcore_axis_name="s")

_M_HI = -65536
_M_LO = 0xFFFF
_RND = 0x7FFF
_ONE = 1


def _rne(b):
    return b + _RND + ((lax.shift_right_logical(b, 16)) & _ONE)


def _unpack16(w):
    lo = lax.bitcast_convert_type(w << 16, jnp.float32)
    hi = lax.bitcast_convert_type(w & _M_HI, jnp.float32)
    return lo, hi


def _pack16(lo, hi):
    lb = _rne(lax.bitcast_convert_type(lo, jnp.int32))
    hb = _rne(lax.bitcast_convert_type(hi, jnp.int32))
    return (hb & _M_HI) | (lax.shift_right_logical(lb, 16) & _M_LO)


def _tc_pack(x):
    lb = _rne(lax.bitcast_convert_type(x[:, :128], jnp.int32))
    hb = _rne(lax.bitcast_convert_type(x[:, 128:], jnp.int32))
    return (hb & _M_HI) | (lax.shift_right_logical(lb, 16) & _M_LO)


def _tc_unpack(w):
    lo = lax.bitcast_convert_type(w << 16, jnp.float32)
    hi = lax.bitcast_convert_type(w & _M_HI, jnp.float32)
    return jnp.concatenate([lo, hi], axis=1)


def _mm_in_body(x_ref, wi_ref, wh_ref, inp_ref, m2_ref):
    inp = jnp.dot(x_ref[...], wi_ref[...], preferred_element_type=jnp.float32)
    inp_ref[...] = inp
    m = jnp.maximum(inp, 0.0).astype(jnp.bfloat16)
    m2_ref[...] = _tc_pack(jnp.dot(m, wh_ref[...],
                                   preferred_element_type=jnp.float32))


def _mm_in(fb, wi, wh_b):
    RB = 2048
    return pl.pallas_call(
        _mm_in_body,
        grid=(E_PAD // RB,),
        in_specs=[pl.BlockSpec((RB, BOND_FDIM), lambda i: (i, 0)),
                  pl.BlockSpec((BOND_FDIM, H), lambda i: (0, 0)),
                  pl.BlockSpec((H, H), lambda i: (0, 0))],
        out_specs=[pl.BlockSpec((RB, H), lambda i: (i, 0)),
                   pl.BlockSpec((RB, H // 2), lambda i: (i, 0))],
        out_shape=[jax.ShapeDtypeStruct((E_PAD, H), jnp.float32),
                   jax.ShapeDtypeStruct((E_PAD, H // 2), jnp.int32)],
    )(fb, wi, wh_b)


def _mm_h_body(inp_ref, g_ref, wh_ref, m2_ref):
    m = jnp.maximum(inp_ref[...] + _tc_unpack(g_ref[...]), 0.0)
    m2_ref[...] = _tc_pack(jnp.dot(m.astype(jnp.bfloat16), wh_ref[...],
                                   preferred_element_type=jnp.float32))


def _mm_h(inp, g, wh_b):
    RB = 2048
    return pl.pallas_call(
        _mm_h_body,
        grid=(E_PAD // RB,),
        in_specs=[pl.BlockSpec((RB, H), lambda i: (i, 0)),
                  pl.BlockSpec((RB, H // 2), lambda i: (i, 0)),
                  pl.BlockSpec((H, H), lambda i: (0, 0))],
        out_specs=pl.BlockSpec((RB, H // 2), lambda i: (i, 0)),
        out_shape=jax.ShapeDtypeStruct((E_PAD, H // 2), jnp.int32),
    )(inp, g, wh_b)


def _relu_add_body(inp_ref, g_ref, out_ref):
    out_ref[...] = jnp.maximum(inp_ref[...] + _tc_unpack(g_ref[...]), 0.0)


def _relu_add(inp, g):
    RB = 4096
    return pl.pallas_call(
        _relu_add_body,
        grid=(E_PAD // RB,),
        in_specs=[pl.BlockSpec((RB, H), lambda i: (i, 0)),
                  pl.BlockSpec((RB, H // 2), lambda i: (i, 0))],
        out_specs=pl.BlockSpec((RB, H), lambda i: (i, 0)),
        out_shape=jax.ShapeDtypeStruct((E_PAD, H), jnp.float32),
    )(inp, g)


def _final_body(fa_ref, am_ref, wo_ref, bo_ref, out_ref):
    wo = wo_ref[...]
    h = jnp.dot(fa_ref[...], wo[:ATOM_FDIM], preferred_element_type=jnp.float32)
    h = h + jnp.dot(am_ref[...], wo[ATOM_FDIM:], preferred_element_type=jnp.float32)
    h = jnp.maximum(h + bo_ref[...], 0.0)
    r = lax.broadcasted_iota(jnp.int32, (N_MOLS, N_PAD), 1) // APM
    m = lax.broadcasted_iota(jnp.int32, (N_MOLS, N_PAD), 0)
    sel = jnp.where(r == m, 1.0 / APM, 0.0)
    out_ref[...] = jnp.dot(sel, h, preferred_element_type=jnp.float32)


def _final(fa, am, wo, bo2):
    return pl.pallas_call(
        _final_body,
        out_shape=jax.ShapeDtypeStruct((N_MOLS, H), jnp.float32),
    )(fa, am, wo, bo2)


@functools.partial(
    pl.kernel,
    mesh=plsc.VectorSubcoreMesh(**_MESH),
    out_type=jax.ShapeDtypeStruct((N_PAD, 128), jnp.int32),
    scratch_types=[
        pltpu.VMEM((CA * MAX_NB,), jnp.int32),
        pltpu.VMEM((CA * MAX_NB, 128), jnp.int32),
        pltpu.VMEM((CA, 128), jnp.int32),
        pltpu.SemaphoreType.DMA,
    ],
)
def _segsum_bf(a2b_hbm, m2_hbm, out_hbm, idx_v, rows_v, acc_v, sem):
    wid = lax.axis_index("s") * NC + lax.axis_index("c")
    base = wid * APW

    def chunk(ci, _):
        a0 = base + ci * CA
        pltpu.sync_copy(a2b_hbm.at[pl.ds(a0 * MAX_NB, CA * MAX_NB)], idx_v)
        pltpu.async_copy(m2_hbm.at[idx_v], rows_v, sem).wait()

        def atom(a, _):
            r0 = a * MAX_NB
            for jj in range(8):
                j = jj * 16
                acc0, acc1 = _unpack16(rows_v[r0, pl.ds(j, 16)])
                for k in range(1, MAX_NB):
                    x0, x1 = _unpack16(rows_v[r0 + k, pl.ds(j, 16)])
                    acc0 = acc0 + x0
                    acc1 = acc1 + x1
                acc_v[a, pl.ds(j, 16)] = _pack16(acc0, acc1)
            return 0

        lax.fori_loop(0, CA, atom, 0)
        pltpu.sync_copy(acc_v, out_hbm.at[pl.ds(a0, CA)])
        return 0

    lax.fori_loop(0, APW // CA, chunk, 0)


@functools.partial(
    pl.kernel,
    mesh=plsc.VectorSubcoreMesh(**_MESH),
    out_type=jax.ShapeDtypeStruct((N_PAD, H), jnp.float32),
    scratch_types=[
        pltpu.VMEM((CA * MAX_NB,), jnp.int32),
        pltpu.VMEM((CA * MAX_NB, H), jnp.float32),
        pltpu.VMEM((CA, H), jnp.float32),
        pltpu.SemaphoreType.DMA,
    ],
)
def _segsum_f32(a2b_hbm, msg_hbm, out_hbm, idx_v, rows_v, acc_v, sem):
    wid = lax.axis_index("s") * NC + lax.axis_index("c")
    base = wid * APW

    def chunk(ci, _):
        a0 = base + ci * CA
        pltpu.sync_copy(a2b_hbm.at[pl.ds(a0 * MAX_NB, CA * MAX_NB)], idx_v)
        pltpu.async_copy(msg_hbm.at[idx_v], rows_v, sem).wait()

        def atom(a, _):
            r0 = a * MAX_NB
            for jj in range(LC):
                j = jj * 16
                acc = rows_v[r0, pl.ds(j, 16)]
                for k in range(1, MAX_NB):
                    acc = acc + rows_v[r0 + k, pl.ds(j, 16)]
                acc_v[a, pl.ds(j, 16)] = acc
            return 0

        lax.fori_loop(0, CA, atom, 0)
        pltpu.sync_copy(acc_v, out_hbm.at[pl.ds(a0, CA)])
        return 0

    lax.fori_loop(0, APW // CA, chunk, 0)


@functools.partial(
    pl.kernel,
    mesh=plsc.VectorSubcoreMesh(**_MESH),
    out_type=jax.ShapeDtypeStruct((E_PAD, 128), jnp.int32),
    scratch_types=[
        pltpu.VMEM((CB,), jnp.int32),
        pltpu.VMEM((CB,), jnp.int32),
        pltpu.VMEM((CB, 128), jnp.int32),
        pltpu.VMEM((CB, 128), jnp.int32),
        pltpu.SemaphoreType.DMA,
        pltpu.SemaphoreType.DMA,
    ],
)
def _gather_sub(b2a_hbm, b2revb_hbm, amsg_hbm, m2_hbm, out_hbm,
                idxa_v, idxr_v, ga_v, gr_v, sema, semr):
    wid = lax.axis_index("s") * NC + lax.axis_index("c")
    base = wid * BPW

    def chunk(ci, _):
        e0 = base + ci * CB
        pltpu.sync_copy(b2a_hbm.at[pl.ds(e0, CB)], idxa_v)
        pltpu.sync_copy(b2revb_hbm.at[pl.ds(e0, CB)], idxr_v)
        cpa = pltpu.async_copy(amsg_hbm.at[idxa_v], ga_v, sema)
        cpr = pltpu.async_copy(m2_hbm.at[idxr_v], gr_v, semr)
        cpa.wait()
        cpr.wait()

        def row(r, _):
            for jj in range(8):
                j = jj * 16
                a0, a1 = _unpack16(ga_v[r, pl.ds(j, 16)])
                b0, b1 = _unpack16(gr_v[r, pl.ds(j, 16)])
                ga_v[r, pl.ds(j, 16)] = _pack16(a0 - b0, a1 - b1)
            return 0

        lax.fori_loop(0, CB, row, 0)
        pltpu.sync_copy(ga_v, out_hbm.at[pl.ds(e0, CB)])
        return 0

    lax.fori_loop(0, BPW // CB, chunk, 0)


def kernel(f_atoms, f_bonds, a2b, b2a, b2revb, W_i, W_h, W_o, b_o):
    a2b_flat = jnp.pad(a2b.astype(jnp.int32).reshape(-1),
                       (0, (N_PAD - N_ATOMS) * MAX_NB))
    b2a_p = jnp.pad(b2a.astype(jnp.int32), (0, E_PAD - N_BONDS))
    b2revb_p = jnp.pad(b2revb.astype(jnp.int32), (0, E_PAD - N_BONDS))
    fb_p = jnp.pad(f_bonds, ((0, E_PAD - N_BONDS), (0, 0)))
    fa_p = jnp.pad(f_atoms, ((0, N_PAD - N_ATOMS), (0, 0)))
    wh_b = W_h.astype(jnp.bfloat16)

    inp, m2p = _mm_in(fb_p, W_i, wh_b)
    g = None
    for it in range(DEPTH - 1):
        amsg2 = _segsum_bf(a2b_flat, m2p)
        g = _gather_sub(b2a_p, b2revb_p, amsg2, m2p)
        if it < DEPTH - 2:
            m2p = _mm_h(inp, g, wh_b)
    msg = _relu_add(inp, g)
    amsg = _segsum_f32(a2b_flat, msg)
    return _final(fa_p, amsg, W_o, b_o.reshape(1, H))


# speedup vs baseline: 2.9994x; 1.2857x over previous
import functools

import jax
import jax.numpy as jnp
from jax import lax
from jax.experimental import pallas as pl
from jax.experimental.pallas import tpu as pltpu
from jax.experimental.pallas import tpu_sc as plsc

N_ATOMS = 10000
N_BONDS = 160000
MAX_NB = 16
ATOM_FDIM = 128
BOND_FDIM = 144
H = 256
DEPTH = 4
N_MOLS = 100
APM = 100

NC, NS = 2, 16
NW = NC * NS
E_PAD = 163840
N_PAD = 10240
BPW = E_PAD // NW
APW = N_PAD // NW
CB = 128
CA = 8
LC = H // 16
LCB = H // 32

_MESH = dict(core_axis_name="c", subcore_axis_name="s")

_M_HI = -65536
_M_LO = 0xFFFF
_RND = 0x7FFF
_ONE = 1


def _rne(b):
    return b + _RND + ((lax.shift_right_logical(b, 16)) & _ONE)


def _unpack16(w):
    lo = lax.bitcast_convert_type(w << 16, jnp.float32)
    hi = lax.bitcast_convert_type(w & _M_HI, jnp.float32)
    return lo, hi


def _pack16(lo, hi):
    lb = _rne(lax.bitcast_convert_type(lo, jnp.int32))
    hb = _rne(lax.bitcast_convert_type(hi, jnp.int32))
    return (hb & _M_HI) | (lax.shift_right_logical(lb, 16) & _M_LO)


def _tc_pack(x):
    lb = _rne(lax.bitcast_convert_type(x[:, :128], jnp.int32))
    hb = _rne(lax.bitcast_convert_type(x[:, 128:], jnp.int32))
    return (hb & _M_HI) | (lax.shift_right_logical(lb, 16) & _M_LO)


def _tc_unpack(w):
    lo = lax.bitcast_convert_type(w << 16, jnp.float32)
    hi = lax.bitcast_convert_type(w & _M_HI, jnp.float32)
    return jnp.concatenate([lo, hi], axis=1)


def _mm_in_body(x_ref, wi_ref, wh_ref, inp_ref, m2_ref):
    inp = jnp.dot(x_ref[...], wi_ref[...], preferred_element_type=jnp.float32)
    inp_ref[...] = inp
    m = jnp.maximum(inp, 0.0).astype(jnp.bfloat16)
    m2_ref[...] = _tc_pack(jnp.dot(m, wh_ref[...],
                                   preferred_element_type=jnp.float32))


def _mm_in(fb, wi, wh_b):
    RB = 2048
    return pl.pallas_call(
        _mm_in_body,
        grid=(E_PAD // RB,),
        in_specs=[pl.BlockSpec((RB, BOND_FDIM), lambda i: (i, 0)),
                  pl.BlockSpec((BOND_FDIM, H), lambda i: (0, 0)),
                  pl.BlockSpec((H, H), lambda i: (0, 0))],
        out_specs=[pl.BlockSpec((RB, H), lambda i: (i, 0)),
                   pl.BlockSpec((RB, H // 2), lambda i: (i, 0))],
        out_shape=[jax.ShapeDtypeStruct((E_PAD, H), jnp.float32),
                   jax.ShapeDtypeStruct((E_PAD, H // 2), jnp.int32)],
    )(fb, wi, wh_b)


def _mm_h_body(inp_ref, g_ref, wh_ref, m2_ref):
    m = jnp.maximum(inp_ref[...] + _tc_unpack(g_ref[...]), 0.0)
    m2_ref[...] = _tc_pack(jnp.dot(m.astype(jnp.bfloat16), wh_ref[...],
                                   preferred_element_type=jnp.float32))


def _mm_h(inp, g, wh_b):
    RB = 2048
    return pl.pallas_call(
        _mm_h_body,
        grid=(E_PAD // RB,),
        in_specs=[pl.BlockSpec((RB, H), lambda i: (i, 0)),
                  pl.BlockSpec((RB, H // 2), lambda i: (i, 0)),
                  pl.BlockSpec((H, H), lambda i: (0, 0))],
        out_specs=pl.BlockSpec((RB, H // 2), lambda i: (i, 0)),
        out_shape=jax.ShapeDtypeStruct((E_PAD, H // 2), jnp.int32),
    )(inp, g, wh_b)


def _relu_add_body(inp_ref, g_ref, out_ref):
    out_ref[...] = jnp.maximum(inp_ref[...] + _tc_unpack(g_ref[...]), 0.0)


def _relu_add(inp, g):
    RB = 4096
    return pl.pallas_call(
        _relu_add_body,
        grid=(E_PAD // RB,),
        in_specs=[pl.BlockSpec((RB, H), lambda i: (i, 0)),
                  pl.BlockSpec((RB, H // 2), lambda i: (i, 0))],
        out_specs=pl.BlockSpec((RB, H), lambda i: (i, 0)),
        out_shape=jax.ShapeDtypeStruct((E_PAD, H), jnp.float32),
    )(inp, g)


def _final_body(fa_ref, am_ref, wo_ref, bo_ref, out_ref):
    wo = wo_ref[...]
    h = jnp.dot(fa_ref[...], wo[:ATOM_FDIM], preferred_element_type=jnp.float32)
    h = h + jnp.dot(am_ref[...], wo[ATOM_FDIM:], preferred_element_type=jnp.float32)
    h = jnp.maximum(h + bo_ref[...], 0.0)
    r = lax.broadcasted_iota(jnp.int32, (N_MOLS, N_PAD), 1) // APM
    m = lax.broadcasted_iota(jnp.int32, (N_MOLS, N_PAD), 0)
    sel = jnp.where(r == m, 1.0 / APM, 0.0)
    out_ref[...] = jnp.dot(sel, h, preferred_element_type=jnp.float32)


def _final(fa, am, wo, bo2):
    return pl.pallas_call(
        _final_body,
        out_shape=jax.ShapeDtypeStruct((N_MOLS, H), jnp.float32),
    )(fa, am, wo, bo2)


NCH_A = APW // CA
NCH_B = BPW // CB


@functools.partial(
    pl.kernel,
    mesh=plsc.VectorSubcoreMesh(**_MESH),
    out_type=jax.ShapeDtypeStruct((N_PAD, 128), jnp.int32),
    scratch_types=[
        pltpu.VMEM((2, CA * MAX_NB), jnp.int32),
        pltpu.VMEM((2, CA * MAX_NB, 128), jnp.int32),
        pltpu.VMEM((2, CA, 128), jnp.int32),
        pltpu.SemaphoreType.DMA,
        pltpu.SemaphoreType.DMA,
        pltpu.SemaphoreType.DMA,
        pltpu.SemaphoreType.DMA,
    ],
)
def _segsum_bf(a2b_hbm, m2_hbm, out_hbm, idx_v, rows_v, acc_v,
               sg0, sg1, ss0, ss1):
    wid = lax.axis_index("s") * NC + lax.axis_index("c")
    base = wid * APW
    SG = (sg0, sg1)
    SS = (ss0, ss1)

    def issue(ci, b):
        a0 = base + ci * CA
        pltpu.sync_copy(a2b_hbm.at[pl.ds(a0 * MAX_NB, CA * MAX_NB)],
                        idx_v.at[b])
        pltpu.async_copy(m2_hbm.at[idx_v.at[b]], rows_v.at[b], SG[b])

    issue(0, 0)

    def outer(gi, _):
        for b in range(2):
            ci = gi * 2 + b
            nb = 1 - b

            @pl.when(ci + 1 < NCH_A)
            def _():
                @pl.when(ci >= 1)
                def _():
                    pltpu.make_async_copy(
                        acc_v.at[nb],
                        out_hbm.at[pl.ds(base + (ci - 1) * CA, CA)],
                        SS[nb]).wait()
                issue(ci + 1, nb)

            pltpu.make_async_copy(m2_hbm.at[idx_v.at[b]], rows_v.at[b],
                                  SG[b]).wait()

            def atom(a, _):
                r0 = a * MAX_NB
                for jj in range(8):
                    j = jj * 16
                    acc0, acc1 = _unpack16(rows_v[b, r0, pl.ds(j, 16)])
                    for k in range(1, MAX_NB):
                        x0, x1 = _unpack16(rows_v[b, r0 + k, pl.ds(j, 16)])
                        acc0 = acc0 + x0
                        acc1 = acc1 + x1
                    acc_v[b, a, pl.ds(j, 16)] = _pack16(acc0, acc1)
                return 0

            lax.fori_loop(0, CA, atom, 0)
            pltpu.async_copy(acc_v.at[b],
                             out_hbm.at[pl.ds(base + ci * CA, CA)], SS[b])
        return 0

    lax.fori_loop(0, NCH_A // 2, outer, 0)
    pltpu.make_async_copy(acc_v.at[0],
                          out_hbm.at[pl.ds(base + (NCH_A - 2) * CA, CA)],
                          SS[0]).wait()
    pltpu.make_async_copy(acc_v.at[1],
                          out_hbm.at[pl.ds(base + (NCH_A - 1) * CA, CA)],
                          SS[1]).wait()


@functools.partial(
    pl.kernel,
    mesh=plsc.VectorSubcoreMesh(**_MESH),
    out_type=jax.ShapeDtypeStruct((N_PAD, H), jnp.float32),
    scratch_types=[
        pltpu.VMEM((2, CA * MAX_NB), jnp.int32),
        pltpu.VMEM((2, CA * MAX_NB, H), jnp.float32),
        pltpu.VMEM((2, CA, H), jnp.float32),
        pltpu.SemaphoreType.DMA,
        pltpu.SemaphoreType.DMA,
        pltpu.SemaphoreType.DMA,
        pltpu.SemaphoreType.DMA,
    ],
)
def _segsum_f32(a2b_hbm, msg_hbm, out_hbm, idx_v, rows_v, acc_v,
                sg0, sg1, ss0, ss1):
    wid = lax.axis_index("s") * NC + lax.axis_index("c")
    base = wid * APW
    SG = (sg0, sg1)
    SS = (ss0, ss1)

    def issue(ci, b):
        a0 = base + ci * CA
        pltpu.sync_copy(a2b_hbm.at[pl.ds(a0 * MAX_NB, CA * MAX_NB)],
                        idx_v.at[b])
        pltpu.async_copy(msg_hbm.at[idx_v.at[b]], rows_v.at[b], SG[b])

    issue(0, 0)

    def outer(gi, _):
        for b in range(2):
            ci = gi * 2 + b
            nb = 1 - b

            @pl.when(ci + 1 < NCH_A)
            def _():
                @pl.when(ci >= 1)
                def _():
                    pltpu.make_async_copy(
                        acc_v.at[nb],
                        out_hbm.at[pl.ds(base + (ci - 1) * CA, CA)],
                        SS[nb]).wait()
                issue(ci + 1, nb)

            pltpu.make_async_copy(msg_hbm.at[idx_v.at[b]], rows_v.at[b],
                                  SG[b]).wait()

            def atom(a, _):
                r0 = a * MAX_NB
                for jj in range(LC):
                    j = jj * 16
                    acc = rows_v[b, r0, pl.ds(j, 16)]
                    for k in range(1, MAX_NB):
                        acc = acc + rows_v[b, r0 + k, pl.ds(j, 16)]
                    acc_v[b, a, pl.ds(j, 16)] = acc
                return 0

            lax.fori_loop(0, CA, atom, 0)
            pltpu.async_copy(acc_v.at[b],
                             out_hbm.at[pl.ds(base + ci * CA, CA)], SS[b])
        return 0

    lax.fori_loop(0, NCH_A // 2, outer, 0)
    pltpu.make_async_copy(acc_v.at[0],
                          out_hbm.at[pl.ds(base + (NCH_A - 2) * CA, CA)],
                          SS[0]).wait()
    pltpu.make_async_copy(acc_v.at[1],
                          out_hbm.at[pl.ds(base + (NCH_A - 1) * CA, CA)],
                          SS[1]).wait()


@functools.partial(
    pl.kernel,
    mesh=plsc.VectorSubcoreMesh(**_MESH),
    out_type=jax.ShapeDtypeStruct((E_PAD, 128), jnp.int32),
    scratch_types=[
        pltpu.VMEM((2, CB), jnp.int32),
        pltpu.VMEM((2, CB), jnp.int32),
        pltpu.VMEM((2, CB, 128), jnp.int32),
        pltpu.VMEM((2, CB, 128), jnp.int32),
        pltpu.SemaphoreType.DMA,
        pltpu.SemaphoreType.DMA,
        pltpu.SemaphoreType.DMA,
        pltpu.SemaphoreType.DMA,
        pltpu.SemaphoreType.DMA,
        pltpu.SemaphoreType.DMA,
    ],
)
def _gather_sub(b2a_hbm, b2revb_hbm, amsg_hbm, m2_hbm, out_hbm,
                idxa_v, idxr_v, ga_v, gr_v,
                sa0, sa1, sr0, sr1, ss0, ss1):
    wid = lax.axis_index("s") * NC + lax.axis_index("c")
    base = wid * BPW
    SA = (sa0, sa1)
    SR = (sr0, sr1)
    SS = (ss0, ss1)

    def issue(ci, b):
        e0 = base + ci * CB
        pltpu.sync_copy(b2a_hbm.at[pl.ds(e0, CB)], idxa_v.at[b])
        pltpu.sync_copy(b2revb_hbm.at[pl.ds(e0, CB)], idxr_v.at[b])
        pltpu.async_copy(amsg_hbm.at[idxa_v.at[b]], ga_v.at[b], SA[b])
        pltpu.async_copy(m2_hbm.at[idxr_v.at[b]], gr_v.at[b], SR[b])

    issue(0, 0)

    def outer(gi, _):
        for b in range(2):
            ci = gi * 2 + b
            nb = 1 - b

            @pl.when(ci + 1 < NCH_B)
            def _():
                @pl.when(ci >= 1)
                def _():
                    pltpu.make_async_copy(
                        ga_v.at[nb],
                        out_hbm.at[pl.ds(base + (ci - 1) * CB, CB)],
                        SS[nb]).wait()
                issue(ci + 1, nb)

            pltpu.make_async_copy(amsg_hbm.at[idxa_v.at[b]], ga_v.at[b],
                                  SA[b]).wait()
            pltpu.make_async_copy(m2_hbm.at[idxr_v.at[b]], gr_v.at[b],
                                  SR[b]).wait()

            def row(r, _):
                for jj in range(8):
                    j = jj * 16
                    a0, a1 = _unpack16(ga_v[b, r, pl.ds(j, 16)])
                    b0, b1 = _unpack16(gr_v[b, r, pl.ds(j, 16)])
                    ga_v[b, r, pl.ds(j, 16)] = _pack16(a0 - b0, a1 - b1)
                return 0

            lax.fori_loop(0, CB, row, 0)
            pltpu.async_copy(ga_v.at[b],
                             out_hbm.at[pl.ds(base + ci * CB, CB)], SS[b])
        return 0

    lax.fori_loop(0, NCH_B // 2, outer, 0)
    pltpu.make_async_copy(ga_v.at[0],
                          out_hbm.at[pl.ds(base + (NCH_B - 2) * CB, CB)],
                          SS[0]).wait()
    pltpu.make_async_copy(ga_v.at[1],
                          out_hbm.at[pl.ds(base + (NCH_B - 1) * CB, CB)],
                          SS[1]).wait()


def kernel(f_atoms, f_bonds, a2b, b2a, b2revb, W_i, W_h, W_o, b_o):
    a2b_flat = jnp.pad(a2b.astype(jnp.int32).reshape(-1),
                       (0, (N_PAD - N_ATOMS) * MAX_NB))
    b2a_p = jnp.pad(b2a.astype(jnp.int32), (0, E_PAD - N_BONDS))
    b2revb_p = jnp.pad(b2revb.astype(jnp.int32), (0, E_PAD - N_BONDS))
    fb_p = jnp.pad(f_bonds, ((0, E_PAD - N_BONDS), (0, 0)))
    fa_p = jnp.pad(f_atoms, ((0, N_PAD - N_ATOMS), (0, 0)))
    wh_b = W_h.astype(jnp.bfloat16)

    inp, m2p = _mm_in(fb_p, W_i, wh_b)
    g = None
    for it in range(DEPTH - 1):
        amsg2 = _segsum_bf(a2b_flat, m2p)
        g = _gather_sub(b2a_p, b2revb_p, amsg2, m2p)
        if it < DEPTH - 2:
            m2p = _mm_h(inp, g, wh_b)
    msg = _relu_add(inp, g)
    amsg = _segsum_f32(a2b_flat, msg)
    return _final(fa_p, amsg, W_o, b_o.reshape(1, H))


# speedup vs baseline: 3.0694x; 1.0233x over previous
import functools

import jax
import jax.numpy as jnp
from jax import lax
from jax.experimental import pallas as pl
from jax.experimental.pallas import tpu as pltpu
from jax.experimental.pallas import tpu_sc as plsc

N_ATOMS = 10000
N_BONDS = 160000
MAX_NB = 16
ATOM_FDIM = 128
BOND_FDIM = 144
H = 256
DEPTH = 4
N_MOLS = 100
APM = 100

NC, NS = 2, 16
NW = NC * NS
E_PAD = 163840
N_PAD = 10240
BPW = E_PAD // NW
APW = N_PAD // NW
CB = 128
CA = 8
LC = H // 16
LCB = H // 32

_MESH = dict(core_axis_name="c", subcore_axis_name="s")

_M_HI = -65536
_M_LO = 0xFFFF
_RND = 0x7FFF
_ONE = 1


def _rne(b):
    return b + _RND + ((lax.shift_right_logical(b, 16)) & _ONE)


def _unpack16(w):
    lo = lax.bitcast_convert_type(w << 16, jnp.float32)
    hi = lax.bitcast_convert_type(w & _M_HI, jnp.float32)
    return lo, hi


def _pack16(lo, hi):
    lb = _rne(lax.bitcast_convert_type(lo, jnp.int32))
    hb = _rne(lax.bitcast_convert_type(hi, jnp.int32))
    return (hb & _M_HI) | (lax.shift_right_logical(lb, 16) & _M_LO)


def _tc_pack(x):
    lb = _rne(lax.bitcast_convert_type(x[:, :128], jnp.int32))
    hb = _rne(lax.bitcast_convert_type(x[:, 128:], jnp.int32))
    return (hb & _M_HI) | (lax.shift_right_logical(lb, 16) & _M_LO)


def _tc_unpack(w):
    lo = lax.bitcast_convert_type(w << 16, jnp.float32)
    hi = lax.bitcast_convert_type(w & _M_HI, jnp.float32)
    return jnp.concatenate([lo, hi], axis=1)


def _mm_in_body(x_ref, wi_ref, wh_ref, inp_ref, m2_ref):
    inp = jnp.dot(x_ref[...], wi_ref[...], preferred_element_type=jnp.float32)
    inp_ref[...] = inp
    m = jnp.maximum(inp, 0.0).astype(jnp.bfloat16)
    m2_ref[...] = _tc_pack(jnp.dot(m, wh_ref[...],
                                   preferred_element_type=jnp.float32))


def _mm_in(fb, wi, wh_b):
    RB = 2048
    return pl.pallas_call(
        _mm_in_body,
        grid=(E_PAD // RB,),
        in_specs=[pl.BlockSpec((RB, BOND_FDIM), lambda i: (i, 0)),
                  pl.BlockSpec((BOND_FDIM, H), lambda i: (0, 0)),
                  pl.BlockSpec((H, H), lambda i: (0, 0))],
        out_specs=[pl.BlockSpec((RB, H), lambda i: (i, 0)),
                   pl.BlockSpec((RB, H // 2), lambda i: (i, 0))],
        out_shape=[jax.ShapeDtypeStruct((E_PAD, H), jnp.float32),
                   jax.ShapeDtypeStruct((E_PAD, H // 2), jnp.int32)],
    )(fb, wi, wh_b)


def _mm_h_body(inp_ref, ga_ref, gr_ref, wh_ref, m2_ref):
    m = jnp.maximum(inp_ref[...] + _tc_unpack(ga_ref[...])
                    - _tc_unpack(gr_ref[...]), 0.0)
    m2_ref[...] = _tc_pack(jnp.dot(m.astype(jnp.bfloat16), wh_ref[...],
                                   preferred_element_type=jnp.float32))


def _mm_h(inp, ga, gr, wh_b):
    RB = 2048
    return pl.pallas_call(
        _mm_h_body,
        grid=(E_PAD // RB,),
        in_specs=[pl.BlockSpec((RB, H), lambda i: (i, 0)),
                  pl.BlockSpec((RB, H // 2), lambda i: (i, 0)),
                  pl.BlockSpec((RB, H // 2), lambda i: (i, 0)),
                  pl.BlockSpec((H, H), lambda i: (0, 0))],
        out_specs=pl.BlockSpec((RB, H // 2), lambda i: (i, 0)),
        out_shape=jax.ShapeDtypeStruct((E_PAD, H // 2), jnp.int32),
    )(inp, ga, gr, wh_b)


def _relu_add_body(inp_ref, ga_ref, gr_ref, out_ref):
    m = jnp.maximum(inp_ref[...] + _tc_unpack(ga_ref[...])
                    - _tc_unpack(gr_ref[...]), 0.0)
    out_ref[...] = _tc_pack(m)


def _relu_add(inp, ga, gr):
    RB = 4096
    return pl.pallas_call(
        _relu_add_body,
        grid=(E_PAD // RB,),
        in_specs=[pl.BlockSpec((RB, H), lambda i: (i, 0)),
                  pl.BlockSpec((RB, H // 2), lambda i: (i, 0)),
                  pl.BlockSpec((RB, H // 2), lambda i: (i, 0))],
        out_specs=pl.BlockSpec((RB, H // 2), lambda i: (i, 0)),
        out_shape=jax.ShapeDtypeStruct((E_PAD, H // 2), jnp.int32),
    )(inp, ga, gr)


def _final_body(fa_ref, am_ref, wo_ref, bo_ref, out_ref):
    wo = wo_ref[...]
    h = jnp.dot(fa_ref[...], wo[:ATOM_FDIM], preferred_element_type=jnp.float32)
    h = h + jnp.dot(_tc_unpack(am_ref[...]), wo[ATOM_FDIM:],
                    preferred_element_type=jnp.float32)
    h = jnp.maximum(h + bo_ref[...], 0.0)
    r = lax.broadcasted_iota(jnp.int32, (N_MOLS, N_PAD), 1) // APM
    m = lax.broadcasted_iota(jnp.int32, (N_MOLS, N_PAD), 0)
    sel = jnp.where(r == m, 1.0 / APM, 0.0)
    out_ref[...] = jnp.dot(sel, h, preferred_element_type=jnp.float32)


def _final(fa, am, wo, bo2):
    return pl.pallas_call(
        _final_body,
        out_shape=jax.ShapeDtypeStruct((N_MOLS, H), jnp.float32),
    )(fa, am, wo, bo2)


NCH_A = APW // CA
NCH_B = BPW // CB


@functools.partial(
    pl.kernel,
    mesh=plsc.VectorSubcoreMesh(**_MESH),
    out_type=jax.ShapeDtypeStruct((N_PAD, 128), jnp.int32),
    scratch_types=[
        pltpu.VMEM((2, CA * MAX_NB), jnp.int32),
        pltpu.VMEM((2, CA * MAX_NB, 128), jnp.int32),
        pltpu.VMEM((2, CA, 128), jnp.int32),
        pltpu.SemaphoreType.DMA,
        pltpu.SemaphoreType.DMA,
        pltpu.SemaphoreType.DMA,
        pltpu.SemaphoreType.DMA,
    ],
)
def _segsum_bf(a2b_hbm, m2_hbm, out_hbm, idx_v, rows_v, acc_v,
               sg0, sg1, ss0, ss1):
    wid = lax.axis_index("s") * NC + lax.axis_index("c")
    base = wid * APW
    SG = (sg0, sg1)
    SS = (ss0, ss1)

    def issue(ci, b):
        a0 = base + ci * CA
        pltpu.sync_copy(a2b_hbm.at[pl.ds(a0 * MAX_NB, CA * MAX_NB)],
                        idx_v.at[b])
        pltpu.async_copy(m2_hbm.at[idx_v.at[b]], rows_v.at[b], SG[b])

    issue(0, 0)

    def outer(gi, _):
        for b in range(2):
            ci = gi * 2 + b
            nb = 1 - b

            @pl.when(ci + 1 < NCH_A)
            def _():
                @pl.when(ci >= 1)
                def _():
                    pltpu.make_async_copy(
                        acc_v.at[nb],
                        out_hbm.at[pl.ds(base + (ci - 1) * CA, CA)],
                        SS[nb]).wait()
                issue(ci + 1, nb)

            pltpu.make_async_copy(m2_hbm.at[idx_v.at[b]], rows_v.at[b],
                                  SG[b]).wait()

            def atom(a, _):
                r0 = a * MAX_NB
                for jj in range(8):
                    j = jj * 16
                    acc0, acc1 = _unpack16(rows_v[b, r0, pl.ds(j, 16)])
                    for k in range(1, MAX_NB):
                        x0, x1 = _unpack16(rows_v[b, r0 + k, pl.ds(j, 16)])
                        acc0 = acc0 + x0
                        acc1 = acc1 + x1
                    acc_v[b, a, pl.ds(j, 16)] = _pack16(acc0, acc1)
                return 0

            lax.fori_loop(0, CA, atom, 0)
            pltpu.async_copy(acc_v.at[b],
                             out_hbm.at[pl.ds(base + ci * CA, CA)], SS[b])
        return 0

    lax.fori_loop(0, NCH_A // 2, outer, 0)
    pltpu.make_async_copy(acc_v.at[0],
                          out_hbm.at[pl.ds(base + (NCH_A - 2) * CA, CA)],
                          SS[0]).wait()
    pltpu.make_async_copy(acc_v.at[1],
                          out_hbm.at[pl.ds(base + (NCH_A - 1) * CA, CA)],
                          SS[1]).wait()


@functools.partial(
    pl.kernel,
    mesh=plsc.VectorSubcoreMesh(**_MESH),
    out_type=[jax.ShapeDtypeStruct((E_PAD, 128), jnp.int32),
              jax.ShapeDtypeStruct((E_PAD, 128), jnp.int32)],
    scratch_types=[
        pltpu.VMEM((2, CB), jnp.int32),
        pltpu.VMEM((2, CB), jnp.int32),
        pltpu.VMEM((2, CB, 128), jnp.int32),
        pltpu.VMEM((2, CB, 128), jnp.int32),
        pltpu.SemaphoreType.DMA,
        pltpu.SemaphoreType.DMA,
        pltpu.SemaphoreType.DMA,
        pltpu.SemaphoreType.DMA,
        pltpu.SemaphoreType.DMA,
        pltpu.SemaphoreType.DMA,
        pltpu.SemaphoreType.DMA,
        pltpu.SemaphoreType.DMA,
    ],
)
def _gather2(b2a_hbm, b2revb_hbm, amsg_hbm, m2_hbm, ga_out, gr_out,
             idxa_v, idxr_v, ga_v, gr_v,
             sa0, sa1, sr0, sr1, ssa0, ssa1, ssr0, ssr1):
    wid = lax.axis_index("s") * NC + lax.axis_index("c")
    base = wid * BPW
    SA = (sa0, sa1)
    SR = (sr0, sr1)
    SSA = (ssa0, ssa1)
    SSR = (ssr0, ssr1)

    def issue(ci, b):
        e0 = base + ci * CB
        pltpu.sync_copy(b2a_hbm.at[pl.ds(e0, CB)], idxa_v.at[b])
        pltpu.sync_copy(b2revb_hbm.at[pl.ds(e0, CB)], idxr_v.at[b])
        pltpu.async_copy(amsg_hbm.at[idxa_v.at[b]], ga_v.at[b], SA[b])
        pltpu.async_copy(m2_hbm.at[idxr_v.at[b]], gr_v.at[b], SR[b])

    issue(0, 0)

    def outer(gi, _):
        for b in range(2):
            ci = gi * 2 + b
            nb = 1 - b

            @pl.when(ci + 1 < NCH_B)
            def _():
                @pl.when(ci >= 1)
                def _():
                    prev = base + (ci - 1) * CB
                    pltpu.make_async_copy(
                        ga_v.at[nb], ga_out.at[pl.ds(prev, CB)],
                        SSA[nb]).wait()
                    pltpu.make_async_copy(
                        gr_v.at[nb], gr_out.at[pl.ds(prev, CB)],
                        SSR[nb]).wait()
                issue(ci + 1, nb)

            pltpu.make_async_copy(amsg_hbm.at[idxa_v.at[b]], ga_v.at[b],
                                  SA[b]).wait()
            pltpu.make_async_copy(m2_hbm.at[idxr_v.at[b]], gr_v.at[b],
                                  SR[b]).wait()
            e0 = base + ci * CB
            pltpu.async_copy(ga_v.at[b], ga_out.at[pl.ds(e0, CB)], SSA[b])
            pltpu.async_copy(gr_v.at[b], gr_out.at[pl.ds(e0, CB)], SSR[b])
        return 0

    lax.fori_loop(0, NCH_B // 2, outer, 0)
    for b, ci in ((0, NCH_B - 2), (1, NCH_B - 1)):
        e0 = base + ci * CB
        pltpu.make_async_copy(ga_v.at[b], ga_out.at[pl.ds(e0, CB)],
                              SSA[b]).wait()
        pltpu.make_async_copy(gr_v.at[b], gr_out.at[pl.ds(e0, CB)],
                              SSR[b]).wait()


def kernel(f_atoms, f_bonds, a2b, b2a, b2revb, W_i, W_h, W_o, b_o):
    a2b_flat = jnp.pad(a2b.astype(jnp.int32).reshape(-1),
                       (0, (N_PAD - N_ATOMS) * MAX_NB))
    b2a_p = jnp.pad(b2a.astype(jnp.int32), (0, E_PAD - N_BONDS))
    b2revb_p = jnp.pad(b2revb.astype(jnp.int32), (0, E_PAD - N_BONDS))
    fb_p = jnp.pad(f_bonds, ((0, E_PAD - N_BONDS), (0, 0)))
    fa_p = jnp.pad(f_atoms, ((0, N_PAD - N_ATOMS), (0, 0)))
    wh_b = W_h.astype(jnp.bfloat16)

    inp, m2p = _mm_in(fb_p, W_i, wh_b)
    ga = gr = None
    for it in range(DEPTH - 1):
        amsg2 = _segsum_bf(a2b_flat, m2p)
        ga, gr = _gather2(b2a_p, b2revb_p, amsg2, m2p)
        if it < DEPTH - 2:
            m2p = _mm_h(inp, ga, gr, wh_b)
    msg_p = _relu_add(inp, ga, gr)
    amsg_p = _segsum_bf(a2b_flat, msg_p)
    return _final(fa_p, amsg_p, W_o, b_o.reshape(1, H))


# speedup vs baseline: 3.1059x; 1.0119x over previous
import functools

import jax
import jax.numpy as jnp
from jax import lax
from jax.experimental import pallas as pl
from jax.experimental.pallas import tpu as pltpu
from jax.experimental.pallas import tpu_sc as plsc

N_ATOMS = 10000
N_BONDS = 160000
MAX_NB = 16
ATOM_FDIM = 128
BOND_FDIM = 144
H = 256
DEPTH = 4
N_MOLS = 100
APM = 100

NC, NS = 2, 16
NW = NC * NS
E_PAD = 163840
N_PAD = 10240
BPW = E_PAD // NW
APW = N_PAD // NW
CB = 128
CA = 8
LC = H // 16
LCB = H // 32

_MESH = dict(core_axis_name="c", subcore_axis_name="s")

_M_HI = -65536
_M_LO = 0xFFFF
_RND = 0x7FFF
_ONE = 1


def _rne(b):
    return b + _RND + ((lax.shift_right_logical(b, 16)) & _ONE)


def _unpack16(w):
    lo = lax.bitcast_convert_type(w << 16, jnp.float32)
    hi = lax.bitcast_convert_type(w & _M_HI, jnp.float32)
    return lo, hi


def _pack16(lo, hi):
    lb = _rne(lax.bitcast_convert_type(lo, jnp.int32))
    hb = _rne(lax.bitcast_convert_type(hi, jnp.int32))
    return (hb & _M_HI) | (lax.shift_right_logical(lb, 16) & _M_LO)


def _tc_pack(x):
    lb = _rne(lax.bitcast_convert_type(x[:, :128], jnp.int32))
    hb = _rne(lax.bitcast_convert_type(x[:, 128:], jnp.int32))
    return (hb & _M_HI) | (lax.shift_right_logical(lb, 16) & _M_LO)


def _tc_unpack(w):
    lo = lax.bitcast_convert_type(w << 16, jnp.float32)
    hi = lax.bitcast_convert_type(w & _M_HI, jnp.float32)
    return jnp.concatenate([lo, hi], axis=1)


def _mm_in_body(x_ref, wi_ref, wh_ref, inp_ref, m2_ref):
    inp = jnp.dot(x_ref[...], wi_ref[...], preferred_element_type=jnp.float32)
    inp_ref[...] = inp
    m = jnp.maximum(inp, 0.0).astype(jnp.bfloat16)
    m2_ref[...] = _tc_pack(jnp.dot(m, wh_ref[...],
                                   preferred_element_type=jnp.float32))


def _mm_in(fb, wi, wh_b):
    RB = 2048
    return pl.pallas_call(
        _mm_in_body,
        grid=(E_PAD // RB,),
        in_specs=[pl.BlockSpec((RB, BOND_FDIM), lambda i: (i, 0)),
                  pl.BlockSpec((BOND_FDIM, H), lambda i: (0, 0)),
                  pl.BlockSpec((H, H), lambda i: (0, 0))],
        out_specs=[pl.BlockSpec((RB, H), lambda i: (i, 0)),
                   pl.BlockSpec((RB, H // 2), lambda i: (i, 0))],
        out_shape=[jax.ShapeDtypeStruct((E_PAD, H), jnp.float32),
                   jax.ShapeDtypeStruct((E_PAD, H // 2), jnp.int32)],
    )(fb, wi, wh_b)


def _mm_h_body(inp_ref, ga_ref, gr_ref, wh_ref, m2_ref):
    m = jnp.maximum(inp_ref[...] + _tc_unpack(ga_ref[...])
                    - _tc_unpack(gr_ref[...]), 0.0)
    m2_ref[...] = _tc_pack(jnp.dot(m.astype(jnp.bfloat16), wh_ref[...],
                                   preferred_element_type=jnp.float32))


def _mm_h(inp, ga, gr, wh_b):
    RB = 2048
    return pl.pallas_call(
        _mm_h_body,
        grid=(E_PAD // RB,),
        in_specs=[pl.BlockSpec((RB, H), lambda i: (i, 0)),
                  pl.BlockSpec((RB, H // 2), lambda i: (i, 0)),
                  pl.BlockSpec((RB, H // 2), lambda i: (i, 0)),
                  pl.BlockSpec((H, H), lambda i: (0, 0))],
        out_specs=pl.BlockSpec((RB, H // 2), lambda i: (i, 0)),
        out_shape=jax.ShapeDtypeStruct((E_PAD, H // 2), jnp.int32),
    )(inp, ga, gr, wh_b)


def _relu_add_body(inp_ref, ga_ref, gr_ref, out_ref):
    m = jnp.maximum(inp_ref[...] + _tc_unpack(ga_ref[...])
                    - _tc_unpack(gr_ref[...]), 0.0)
    out_ref[...] = _tc_pack(m)


def _relu_add(inp, ga, gr):
    RB = 4096
    return pl.pallas_call(
        _relu_add_body,
        grid=(E_PAD // RB,),
        in_specs=[pl.BlockSpec((RB, H), lambda i: (i, 0)),
                  pl.BlockSpec((RB, H // 2), lambda i: (i, 0)),
                  pl.BlockSpec((RB, H // 2), lambda i: (i, 0))],
        out_specs=pl.BlockSpec((RB, H // 2), lambda i: (i, 0)),
        out_shape=jax.ShapeDtypeStruct((E_PAD, H // 2), jnp.int32),
    )(inp, ga, gr)


def _final_body(fa_ref, am_ref, wo_ref, bo_ref, out_ref):
    wo = wo_ref[...]
    h = jnp.dot(fa_ref[...], wo[:ATOM_FDIM], preferred_element_type=jnp.float32)
    h = h + jnp.dot(_tc_unpack(am_ref[...]), wo[ATOM_FDIM:],
                    preferred_element_type=jnp.float32)
    h = jnp.maximum(h + bo_ref[...], 0.0)
    r = lax.broadcasted_iota(jnp.int32, (N_MOLS, N_PAD), 1) // APM
    m = lax.broadcasted_iota(jnp.int32, (N_MOLS, N_PAD), 0)
    sel = jnp.where(r == m, 1.0 / APM, 0.0)
    out_ref[...] = jnp.dot(sel, h, preferred_element_type=jnp.float32)


def _final(fa, am, wo, bo2):
    return pl.pallas_call(
        _final_body,
        out_shape=jax.ShapeDtypeStruct((N_MOLS, H), jnp.float32),
    )(fa, am, wo, bo2)


NCH_A = APW // CA
NCH_B = BPW // CB


@functools.partial(
    pl.kernel,
    mesh=plsc.VectorSubcoreMesh(**_MESH),
    out_type=jax.ShapeDtypeStruct((N_PAD, 128), jnp.int32),
    scratch_types=[
        pltpu.VMEM((APW * MAX_NB,), jnp.int32),
        pltpu.VMEM((2, CA * MAX_NB, 128), jnp.int32),
        pltpu.VMEM((2, CA, 128), jnp.int32),
        pltpu.SemaphoreType.DMA,
        pltpu.SemaphoreType.DMA,
        pltpu.SemaphoreType.DMA,
        pltpu.SemaphoreType.DMA,
    ],
)
def _segsum_bf(a2b_hbm, m2_hbm, out_hbm, idx_v, rows_v, acc_v,
               sg0, sg1, ss0, ss1):
    wid = lax.axis_index("s") * NC + lax.axis_index("c")
    base = wid * APW
    SG = (sg0, sg1)
    SS = (ss0, ss1)

    pltpu.sync_copy(a2b_hbm.at[pl.ds(base * MAX_NB, APW * MAX_NB)], idx_v)

    def issue(ci, b):
        pltpu.async_copy(
            m2_hbm.at[idx_v.at[pl.ds(ci * (CA * MAX_NB), CA * MAX_NB)]],
            rows_v.at[b], SG[b])

    issue(0, 0)

    def outer(gi, _):
        for b in range(2):
            ci = gi * 2 + b
            nb = 1 - b

            @pl.when(ci + 1 < NCH_A)
            def _():
                @pl.when(ci >= 1)
                def _():
                    pltpu.make_async_copy(
                        acc_v.at[nb],
                        out_hbm.at[pl.ds(base + (ci - 1) * CA, CA)],
                        SS[nb]).wait()
                issue(ci + 1, nb)

            pltpu.make_async_copy(
                m2_hbm.at[idx_v.at[pl.ds(ci * (CA * MAX_NB), CA * MAX_NB)]],
                rows_v.at[b], SG[b]).wait()

            def atom(a, _):
                r0 = a * MAX_NB
                for jj in range(8):
                    j = jj * 16
                    ws = [rows_v[b, r0 + k, pl.ds(j, 16)]
                          for k in range(MAX_NB)]
                    los = [lax.bitcast_convert_type(w << 16, jnp.float32)
                           for w in ws]
                    his = [lax.bitcast_convert_type(w & _M_HI, jnp.float32)
                           for w in ws]
                    while len(los) > 1:
                        los = [los[i] + los[i + 1]
                               for i in range(0, len(los), 2)]
                        his = [his[i] + his[i + 1]
                               for i in range(0, len(his), 2)]
                    acc_v[b, a, pl.ds(j, 16)] = _pack16(los[0], his[0])
                return 0

            lax.fori_loop(0, CA, atom, 0)
            pltpu.async_copy(acc_v.at[b],
                             out_hbm.at[pl.ds(base + ci * CA, CA)], SS[b])
        return 0

    lax.fori_loop(0, NCH_A // 2, outer, 0)
    pltpu.make_async_copy(acc_v.at[0],
                          out_hbm.at[pl.ds(base + (NCH_A - 2) * CA, CA)],
                          SS[0]).wait()
    pltpu.make_async_copy(acc_v.at[1],
                          out_hbm.at[pl.ds(base + (NCH_A - 1) * CA, CA)],
                          SS[1]).wait()


@functools.partial(
    pl.kernel,
    mesh=plsc.VectorSubcoreMesh(**_MESH),
    out_type=[jax.ShapeDtypeStruct((E_PAD, 128), jnp.int32),
              jax.ShapeDtypeStruct((E_PAD, 128), jnp.int32)],
    scratch_types=[
        pltpu.VMEM((BPW,), jnp.int32),
        pltpu.VMEM((BPW,), jnp.int32),
        pltpu.VMEM((2, CB, 128), jnp.int32),
        pltpu.VMEM((2, CB, 128), jnp.int32),
        pltpu.SemaphoreType.DMA,
        pltpu.SemaphoreType.DMA,
        pltpu.SemaphoreType.DMA,
        pltpu.SemaphoreType.DMA,
        pltpu.SemaphoreType.DMA,
        pltpu.SemaphoreType.DMA,
        pltpu.SemaphoreType.DMA,
        pltpu.SemaphoreType.DMA,
    ],
)
def _gather2(b2a_hbm, b2revb_hbm, amsg_hbm, m2_hbm, ga_out, gr_out,
             idxa_v, idxr_v, ga_v, gr_v,
             sa0, sa1, sr0, sr1, ssa0, ssa1, ssr0, ssr1):
    wid = lax.axis_index("s") * NC + lax.axis_index("c")
    base = wid * BPW
    SA = (sa0, sa1)
    SR = (sr0, sr1)
    SSA = (ssa0, ssa1)
    SSR = (ssr0, ssr1)

    pltpu.sync_copy(b2a_hbm.at[pl.ds(base, BPW)], idxa_v)
    pltpu.sync_copy(b2revb_hbm.at[pl.ds(base, BPW)], idxr_v)

    def issue(ci, b):
        pltpu.async_copy(amsg_hbm.at[idxa_v.at[pl.ds(ci * CB, CB)]],
                         ga_v.at[b], SA[b])
        pltpu.async_copy(m2_hbm.at[idxr_v.at[pl.ds(ci * CB, CB)]],
                         gr_v.at[b], SR[b])

    issue(0, 0)

    def outer(gi, _):
        for b in range(2):
            ci = gi * 2 + b
            nb = 1 - b

            @pl.when(ci + 1 < NCH_B)
            def _():
                @pl.when(ci >= 1)
                def _():
                    prev = base + (ci - 1) * CB
                    pltpu.make_async_copy(
                        ga_v.at[nb], ga_out.at[pl.ds(prev, CB)],
                        SSA[nb]).wait()
                    pltpu.make_async_copy(
                        gr_v.at[nb], gr_out.at[pl.ds(prev, CB)],
                        SSR[nb]).wait()
                issue(ci + 1, nb)

            pltpu.make_async_copy(amsg_hbm.at[idxa_v.at[pl.ds(ci * CB, CB)]],
                                  ga_v.at[b], SA[b]).wait()
            pltpu.make_async_copy(m2_hbm.at[idxr_v.at[pl.ds(ci * CB, CB)]],
                                  gr_v.at[b], SR[b]).wait()
            e0 = base + ci * CB
            pltpu.async_copy(ga_v.at[b], ga_out.at[pl.ds(e0, CB)], SSA[b])
            pltpu.async_copy(gr_v.at[b], gr_out.at[pl.ds(e0, CB)], SSR[b])
        return 0

    lax.fori_loop(0, NCH_B // 2, outer, 0)
    for b, ci in ((0, NCH_B - 2), (1, NCH_B - 1)):
        e0 = base + ci * CB
        pltpu.make_async_copy(ga_v.at[b], ga_out.at[pl.ds(e0, CB)],
                              SSA[b]).wait()
        pltpu.make_async_copy(gr_v.at[b], gr_out.at[pl.ds(e0, CB)],
                              SSR[b]).wait()


def kernel(f_atoms, f_bonds, a2b, b2a, b2revb, W_i, W_h, W_o, b_o):
    a2b_flat = jnp.pad(a2b.astype(jnp.int32).reshape(-1),
                       (0, (N_PAD - N_ATOMS) * MAX_NB))
    b2a_p = jnp.pad(b2a.astype(jnp.int32), (0, E_PAD - N_BONDS))
    b2revb_p = jnp.pad(b2revb.astype(jnp.int32), (0, E_PAD - N_BONDS))
    fb_p = jnp.pad(f_bonds, ((0, E_PAD - N_BONDS), (0, 0)))
    fa_p = jnp.pad(f_atoms, ((0, N_PAD - N_ATOMS), (0, 0)))
    wh_b = W_h.astype(jnp.bfloat16)

    inp, m2p = _mm_in(fb_p, W_i, wh_b)
    ga = gr = None
    for it in range(DEPTH - 1):
        amsg2 = _segsum_bf(a2b_flat, m2p)
        ga, gr = _gather2(b2a_p, b2revb_p, amsg2, m2p)
        if it < DEPTH - 2:
            m2p = _mm_h(inp, ga, gr, wh_b)
    msg_p = _relu_add(inp, ga, gr)
    amsg_p = _segsum_bf(a2b_flat, msg_p)
    return _final(fa_p, amsg_p, W_o, b_o.reshape(1, H))


# speedup vs baseline: 3.1694x; 1.0205x over previous
import functools

import jax
import jax.numpy as jnp
from jax import lax
from jax.experimental import pallas as pl
from jax.experimental.pallas import tpu as pltpu
from jax.experimental.pallas import tpu_sc as plsc

N_ATOMS = 10000
N_BONDS = 160000
MAX_NB = 16
ATOM_FDIM = 128
BOND_FDIM = 144
H = 256
DEPTH = 4
N_MOLS = 100
APM = 100

NC, NS = 2, 16
NW = NC * NS
E_PAD = 163840
N_PAD = 10240
BPW = E_PAD // NW
APW = N_PAD // NW
CB = 128
CA = 8
LC = H // 16
LCB = H // 32

_MESH = dict(core_axis_name="c", subcore_axis_name="s")

_M_HI = -65536
_M_LO = 0xFFFF
_RND = 0x7FFF
_ONE = 1


def _rne(b):
    return b + _RND + ((lax.shift_right_logical(b, 16)) & _ONE)


def _unpack16(w):
    lo = lax.bitcast_convert_type(w << 16, jnp.float32)
    hi = lax.bitcast_convert_type(w & _M_HI, jnp.float32)
    return lo, hi


def _pack16(lo, hi):
    lb = _rne(lax.bitcast_convert_type(lo, jnp.int32))
    hb = _rne(lax.bitcast_convert_type(hi, jnp.int32))
    return (hb & _M_HI) | (lax.shift_right_logical(lb, 16) & _M_LO)


def _tc_pack(x):
    lb = _rne(lax.bitcast_convert_type(x[:, :128], jnp.int32))
    hb = _rne(lax.bitcast_convert_type(x[:, 128:], jnp.int32))
    return (hb & _M_HI) | (lax.shift_right_logical(lb, 16) & _M_LO)


def _tc_unpack(w):
    lo = lax.bitcast_convert_type(w << 16, jnp.float32)
    hi = lax.bitcast_convert_type(w & _M_HI, jnp.float32)
    return jnp.concatenate([lo, hi], axis=1)


def _mm_in_body(x_ref, wi_ref, wh_ref, inp_ref, m2_ref):
    inp = jnp.dot(x_ref[...], wi_ref[...], preferred_element_type=jnp.float32)
    inp_ref[...] = _tc_pack(inp)
    m = jnp.maximum(inp, 0.0).astype(jnp.bfloat16)
    m2_ref[...] = _tc_pack(jnp.dot(m, wh_ref[...],
                                   preferred_element_type=jnp.float32))


def _mm_in(fb, wi, wh_b):
    RB = 2048
    return pl.pallas_call(
        _mm_in_body,
        grid=(E_PAD // RB,),
        in_specs=[pl.BlockSpec((RB, BOND_FDIM), lambda i: (i, 0)),
                  pl.BlockSpec((BOND_FDIM, H), lambda i: (0, 0)),
                  pl.BlockSpec((H, H), lambda i: (0, 0))],
        out_specs=[pl.BlockSpec((RB, H // 2), lambda i: (i, 0)),
                   pl.BlockSpec((RB, H // 2), lambda i: (i, 0))],
        out_shape=[jax.ShapeDtypeStruct((E_PAD, H // 2), jnp.int32),
                   jax.ShapeDtypeStruct((E_PAD, H // 2), jnp.int32)],
    )(fb, wi, wh_b)


def _mm_h_body(inp_ref, ga_ref, gr_ref, wh_ref, m2_ref):
    m = jnp.maximum(_tc_unpack(inp_ref[...]) + _tc_unpack(ga_ref[...])
                    - _tc_unpack(gr_ref[...]), 0.0)
    m2_ref[...] = _tc_pack(jnp.dot(m.astype(jnp.bfloat16), wh_ref[...],
                                   preferred_element_type=jnp.float32))


def _mm_h(inp, ga, gr, wh_b):
    RB = 2048
    return pl.pallas_call(
        _mm_h_body,
        grid=(E_PAD // RB,),
        in_specs=[pl.BlockSpec((RB, H // 2), lambda i: (i, 0)),
                  pl.BlockSpec((RB, H // 2), lambda i: (i, 0)),
                  pl.BlockSpec((RB, H // 2), lambda i: (i, 0)),
                  pl.BlockSpec((H, H), lambda i: (0, 0))],
        out_specs=pl.BlockSpec((RB, H // 2), lambda i: (i, 0)),
        out_shape=jax.ShapeDtypeStruct((E_PAD, H // 2), jnp.int32),
    )(inp, ga, gr, wh_b)


def _relu_add_body(inp_ref, ga_ref, gr_ref, out_ref):
    m = jnp.maximum(_tc_unpack(inp_ref[...]) + _tc_unpack(ga_ref[...])
                    - _tc_unpack(gr_ref[...]), 0.0)
    out_ref[...] = _tc_pack(m)


def _relu_add(inp, ga, gr):
    RB = 4096
    return pl.pallas_call(
        _relu_add_body,
        grid=(E_PAD // RB,),
        in_specs=[pl.BlockSpec((RB, H // 2), lambda i: (i, 0)),
                  pl.BlockSpec((RB, H // 2), lambda i: (i, 0)),
                  pl.BlockSpec((RB, H // 2), lambda i: (i, 0))],
        out_specs=pl.BlockSpec((RB, H // 2), lambda i: (i, 0)),
        out_shape=jax.ShapeDtypeStruct((E_PAD, H // 2), jnp.int32),
    )(inp, ga, gr)


def _final_body(fa_ref, am_ref, wo_ref, bo_ref, out_ref):
    wo = wo_ref[...]
    h = jnp.dot(fa_ref[...], wo[:ATOM_FDIM], preferred_element_type=jnp.float32)
    h = h + jnp.dot(_tc_unpack(am_ref[...]), wo[ATOM_FDIM:],
                    preferred_element_type=jnp.float32)
    h = jnp.maximum(h + bo_ref[...], 0.0)
    r = lax.broadcasted_iota(jnp.int32, (N_MOLS, N_PAD), 1) // APM
    m = lax.broadcasted_iota(jnp.int32, (N_MOLS, N_PAD), 0)
    sel = jnp.where(r == m, 1.0 / APM, 0.0)
    out_ref[...] = jnp.dot(sel, h, preferred_element_type=jnp.float32)


def _final(fa, am, wo, bo2):
    return pl.pallas_call(
        _final_body,
        out_shape=jax.ShapeDtypeStruct((N_MOLS, H), jnp.float32),
    )(fa, am, wo, bo2)


NCH_A = APW // CA
NCH_B = BPW // CB


@functools.partial(
    pl.kernel,
    mesh=plsc.VectorSubcoreMesh(**_MESH),
    out_type=jax.ShapeDtypeStruct((N_PAD, 128), jnp.int32),
    scratch_types=[
        pltpu.VMEM((APW * MAX_NB,), jnp.int32),
        pltpu.VMEM((2, CA * MAX_NB, 128), jnp.int32),
        pltpu.VMEM((2, CA, 128), jnp.int32),
        pltpu.SemaphoreType.DMA,
        pltpu.SemaphoreType.DMA,
        pltpu.SemaphoreType.DMA,
        pltpu.SemaphoreType.DMA,
    ],
)
def _segsum_bf(a2b_hbm, m2_hbm, out_hbm, idx_v, rows_v, acc_v,
               sg0, sg1, ss0, ss1):
    wid = lax.axis_index("s") * NC + lax.axis_index("c")
    base = wid * APW
    SG = (sg0, sg1)
    SS = (ss0, ss1)

    pltpu.sync_copy(a2b_hbm.at[pl.ds(base * MAX_NB, APW * MAX_NB)], idx_v)

    def issue(ci, b):
        pltpu.async_copy(
            m2_hbm.at[idx_v.at[pl.ds(ci * (CA * MAX_NB), CA * MAX_NB)]],
            rows_v.at[b], SG[b])

    issue(0, 0)

    def outer(gi, _):
        for b in range(2):
            ci = gi * 2 + b
            nb = 1 - b

            @pl.when(ci + 1 < NCH_A)
            def _():
                @pl.when(ci >= 1)
                def _():
                    pltpu.make_async_copy(
                        acc_v.at[nb],
                        out_hbm.at[pl.ds(base + (ci - 1) * CA, CA)],
                        SS[nb]).wait()
                issue(ci + 1, nb)

            pltpu.make_async_copy(
                m2_hbm.at[idx_v.at[pl.ds(ci * (CA * MAX_NB), CA * MAX_NB)]],
                rows_v.at[b], SG[b]).wait()

            def atom(a, _):
                r0 = a * MAX_NB
                for jj in range(8):
                    j = jj * 16
                    ws = [rows_v[b, r0 + k, pl.ds(j, 16)]
                          for k in range(MAX_NB)]
                    los = [lax.bitcast_convert_type(w << 16, jnp.float32)
                           for w in ws]
                    his = [lax.bitcast_convert_type(w & _M_HI, jnp.float32)
                           for w in ws]
                    while len(los) > 1:
                        los = [los[i] + los[i + 1]
                               for i in range(0, len(los), 2)]
                        his = [his[i] + his[i + 1]
                               for i in range(0, len(his), 2)]
                    acc_v[b, a, pl.ds(j, 16)] = _pack16(los[0], his[0])
                return 0

            lax.fori_loop(0, CA, atom, 0)
            pltpu.async_copy(acc_v.at[b],
                             out_hbm.at[pl.ds(base + ci * CA, CA)], SS[b])
        return 0

    lax.fori_loop(0, NCH_A // 2, outer, 0)
    pltpu.make_async_copy(acc_v.at[0],
                          out_hbm.at[pl.ds(base + (NCH_A - 2) * CA, CA)],
                          SS[0]).wait()
    pltpu.make_async_copy(acc_v.at[1],
                          out_hbm.at[pl.ds(base + (NCH_A - 1) * CA, CA)],
                          SS[1]).wait()


@functools.partial(
    pl.kernel,
    mesh=plsc.VectorSubcoreMesh(**_MESH),
    out_type=[jax.ShapeDtypeStruct((E_PAD, 128), jnp.int32),
              jax.ShapeDtypeStruct((E_PAD, 128), jnp.int32)],
    scratch_types=[
        pltpu.VMEM((BPW,), jnp.int32),
        pltpu.VMEM((BPW,), jnp.int32),
        pltpu.VMEM((2, CB, 128), jnp.int32),
        pltpu.VMEM((2, CB, 128), jnp.int32),
        pltpu.SemaphoreType.DMA,
        pltpu.SemaphoreType.DMA,
        pltpu.SemaphoreType.DMA,
        pltpu.SemaphoreType.DMA,
        pltpu.SemaphoreType.DMA,
        pltpu.SemaphoreType.DMA,
        pltpu.SemaphoreType.DMA,
        pltpu.SemaphoreType.DMA,
    ],
)
def _gather2(b2a_hbm, b2revb_hbm, amsg_hbm, m2_hbm, ga_out, gr_out,
             idxa_v, idxr_v, ga_v, gr_v,
             sa0, sa1, sr0, sr1, ssa0, ssa1, ssr0, ssr1):
    wid = lax.axis_index("s") * NC + lax.axis_index("c")
    base = wid * BPW
    SA = (sa0, sa1)
    SR = (sr0, sr1)
    SSA = (ssa0, ssa1)
    SSR = (ssr0, ssr1)

    pltpu.sync_copy(b2a_hbm.at[pl.ds(base, BPW)], idxa_v)
    pltpu.sync_copy(b2revb_hbm.at[pl.ds(base, BPW)], idxr_v)

    def issue(ci, b):
        pltpu.async_copy(amsg_hbm.at[idxa_v.at[pl.ds(ci * CB, CB)]],
                         ga_v.at[b], SA[b])
        pltpu.async_copy(m2_hbm.at[idxr_v.at[pl.ds(ci * CB, CB)]],
                         gr_v.at[b], SR[b])

    issue(0, 0)

    def outer(gi, _):
        for b in range(2):
            ci = gi * 2 + b
            nb = 1 - b

            @pl.when(ci + 1 < NCH_B)
            def _():
                @pl.when(ci >= 1)
                def _():
                    prev = base + (ci - 1) * CB
                    pltpu.make_async_copy(
                        ga_v.at[nb], ga_out.at[pl.ds(prev, CB)],
                        SSA[nb]).wait()
                    pltpu.make_async_copy(
                        gr_v.at[nb], gr_out.at[pl.ds(prev, CB)],
                        SSR[nb]).wait()
                issue(ci + 1, nb)

            pltpu.make_async_copy(amsg_hbm.at[idxa_v.at[pl.ds(ci * CB, CB)]],
                                  ga_v.at[b], SA[b]).wait()
            pltpu.make_async_copy(m2_hbm.at[idxr_v.at[pl.ds(ci * CB, CB)]],
                                  gr_v.at[b], SR[b]).wait()
            e0 = base + ci * CB
            pltpu.async_copy(ga_v.at[b], ga_out.at[pl.ds(e0, CB)], SSA[b])
            pltpu.async_copy(gr_v.at[b], gr_out.at[pl.ds(e0, CB)], SSR[b])
        return 0

    lax.fori_loop(0, NCH_B // 2, outer, 0)
    for b, ci in ((0, NCH_B - 2), (1, NCH_B - 1)):
        e0 = base + ci * CB
        pltpu.make_async_copy(ga_v.at[b], ga_out.at[pl.ds(e0, CB)],
                              SSA[b]).wait()
        pltpu.make_async_copy(gr_v.at[b], gr_out.at[pl.ds(e0, CB)],
                              SSR[b]).wait()


def kernel(f_atoms, f_bonds, a2b, b2a, b2revb, W_i, W_h, W_o, b_o):
    a2b_flat = jnp.pad(a2b.astype(jnp.int32).reshape(-1),
                       (0, (N_PAD - N_ATOMS) * MAX_NB))
    b2a_p = jnp.pad(b2a.astype(jnp.int32), (0, E_PAD - N_BONDS))
    b2revb_p = jnp.pad(b2revb.astype(jnp.int32), (0, E_PAD - N_BONDS))
    fb_p = jnp.pad(f_bonds, ((0, E_PAD - N_BONDS), (0, 0)))
    fa_p = jnp.pad(f_atoms, ((0, N_PAD - N_ATOMS), (0, 0)))
    wh_b = W_h.astype(jnp.bfloat16)

    inp, m2p = _mm_in(fb_p, W_i, wh_b)
    ga = gr = None
    for it in range(DEPTH - 1):
        amsg2 = _segsum_bf(a2b_flat, m2p)
        ga, gr = _gather2(b2a_p, b2revb_p, amsg2, m2p)
        if it < DEPTH - 2:
            m2p = _mm_h(inp, ga, gr, wh_b)
    msg_p = _relu_add(inp, ga, gr)
    amsg_p = _segsum_bf(a2b_flat, msg_p)
    return _final(fa_p, amsg_p, W_o, b_o.reshape(1, H))
